# baseline (speedup 1.0000x reference)
import jax
import jax.numpy as jnp
from jax.experimental import pallas as pl
from jax.experimental.pallas import tpu as pltpu

_TOKENS = 32768
_FEAT = 4096
_EXPERTS = 64
_CH = 512
_SUB = 16
_ROWS = _CH // _SUB
_NBUF = 6
_NCH = _TOKENS // _CH


def _router_body(x_hbm, w_ref, b_ref, w_hbm, l_hbm, buf, wtb, wstage, lstage,
                 in_sems, wout_sems, lout_sems):
    def in_copies(i, slot):
        return [pltpu.make_async_copy(
            x_hbm.at[pl.ds(i * _CH + j * _ROWS, _ROWS), :],
            buf.at[slot, pl.ds(j * _ROWS, _ROWS), :],
            in_sems.at[slot]) for j in range(_SUB)]

    def out_copies(i, slot):
        rows = pl.ds(i * _CH, _CH)
        return (
            pltpu.make_async_copy(wstage.at[slot], w_hbm.at[rows, :],
                                  wout_sems.at[slot]),
            pltpu.make_async_copy(lstage.at[slot], l_hbm.at[rows, :],
                                  lout_sems.at[slot]),
        )

    for j in range(_NBUF):
        for c in in_copies(j, j):
            c.start()

    wtb[...] = jnp.transpose(w_ref[...]).astype(jnp.bfloat16)

    w = wtb[...]
    bias = b_ref[...]
    for i in range(_NCH):
        slot = i % _NBUF
        pltpu.make_async_copy(
            x_hbm.at[pl.ds(i * _CH, _CH), :], buf.at[slot],
            in_sems.at[slot]).wait()
        xb = buf[slot].astype(jnp.bfloat16)
        logits = jnp.dot(xb, w, preferred_element_type=jnp.float32) + bias
        m = jnp.max(logits, axis=1, keepdims=True)
        e = jnp.exp(logits - m)
        weights = e / jnp.sum(e, axis=1, keepdims=True)
        if i >= _NBUF:
            wprev, lprev = out_copies(i - _NBUF, slot)
            wprev.wait()
            lprev.wait()
        lstage[slot] = logits
        wstage[slot] = weights
        wcur, lcur = out_copies(i, slot)
        wcur.start()
        lcur.start()
        if i + _NBUF < _NCH:
            for c in in_copies(i + _NBUF, slot):
                c.start()

    for i in range(_NCH - _NBUF, _NCH):
        wlast, llast = out_copies(i, i % _NBUF)
        wlast.wait()
        llast.wait()


def kernel(x, W, b):
    b2 = b.reshape(1, _EXPERTS)
    weights, logits = pl.pallas_call(
        _router_body,
        in_specs=[
            pl.BlockSpec(memory_space=pl.ANY),
            pl.BlockSpec(memory_space=pltpu.VMEM),
            pl.BlockSpec(memory_space=pltpu.VMEM),
        ],
        out_specs=[
            pl.BlockSpec(memory_space=pl.ANY),
            pl.BlockSpec(memory_space=pl.ANY),
        ],
        out_shape=[
            jax.ShapeDtypeStruct((_TOKENS, _EXPERTS), jnp.float32),
            jax.ShapeDtypeStruct((_TOKENS, _EXPERTS), jnp.float32),
        ],
        scratch_shapes=[
            pltpu.VMEM((_NBUF, _CH, _FEAT), jnp.float32),
            pltpu.VMEM((_FEAT, _EXPERTS), jnp.bfloat16),
            pltpu.VMEM((_NBUF, _CH, _EXPERTS), jnp.float32),
            pltpu.VMEM((_NBUF, _CH, _EXPERTS), jnp.float32),
            pltpu.SemaphoreType.DMA((_NBUF,)),
            pltpu.SemaphoreType.DMA((_NBUF,)),
            pltpu.SemaphoreType.DMA((_NBUF,)),
        ],
        compiler_params=pltpu.CompilerParams(
            vmem_limit_bytes=63 * 1024 * 1024,
        ),
    )(x, W, b2)
    return (weights, logits)


# speedup vs baseline: 1.0104x; 1.0104x over previous
import jax
import jax.numpy as jnp
from jax.experimental import pallas as pl

_TOKENS = 32768
_FEAT = 4096
_EXPERTS = 64
_BT = 1024


def _router_body(x_ref, w_ref, b_ref, weights_ref, logits_ref):
    xb = x_ref[...].astype(jnp.bfloat16)
    logits = jax.lax.dot_general(
        xb, w_ref[...], (((1,), (1,)), ((), ())),
        preferred_element_type=jnp.float32)
    logits = logits + b_ref[...]
    logits_ref[...] = logits
    m = jnp.max(logits, axis=1, keepdims=True)
    e = jnp.exp(logits - m)
    weights_ref[...] = e / jnp.sum(e, axis=1, keepdims=True)


def kernel(x, W, b):
    wb = W.astype(jnp.bfloat16)
    b2 = b.reshape(1, _EXPERTS)
    grid = (_TOKENS // _BT,)
    weights, logits = pl.pallas_call(
        _router_body,
        grid=grid,
        in_specs=[
            pl.BlockSpec((_BT, _FEAT), lambda i: (i, 0)),
            pl.BlockSpec((_EXPERTS, _FEAT), lambda i: (0, 0)),
            pl.BlockSpec((1, _EXPERTS), lambda i: (0, 0)),
        ],
        out_specs=[
            pl.BlockSpec((_BT, _EXPERTS), lambda i: (i, 0)),
            pl.BlockSpec((_BT, _EXPERTS), lambda i: (i, 0)),
        ],
        out_shape=[
            jax.ShapeDtypeStruct((_TOKENS, _EXPERTS), jnp.float32),
            jax.ShapeDtypeStruct((_TOKENS, _EXPERTS), jnp.float32),
        ],
    )(x, wb, b2)
    return (weights, logits)
